# baseline (speedup 1.0000x reference)
import functools

import jax
import jax.numpy as jnp
from jax import lax
from jax.experimental import pallas as pl
from jax.experimental.pallas import tpu as pltpu
from jax.experimental.pallas import tpu_sc as plsc

NUM_E = 8192
DIM = 32
ROW_BLOCK = 256
WINDOW = 1024

_NC = 2
_NS = 16
_NW = _NC * _NS
_CHUNK = 128
_CHUNKS_PER_W = NUM_E // (_NW * _CHUNK)


def _argmin_body(x_ref, w_ref, a_ref, b_ref, idx_ref):
    x = x_ref[...]
    w = w_ref[...]
    xb = (2.0 * x).astype(jnp.bfloat16)
    wb = w.astype(jnp.bfloat16)
    m = lax.dot_general(xb, wb, (((1,), (1,)), ((), ())),
                        preferred_element_type=jnp.float32)
    d = (a_ref[...] + b_ref[...]) - m

    lvs, ivs = [], []
    col = lax.broadcasted_iota(jnp.int32, (ROW_BLOCK, WINDOW), 1)
    for wnd in range(NUM_E // WINDOW):
        dw = d[:, wnd * WINDOW:(wnd + 1) * WINDOW]
        mn = jnp.min(dw, axis=-1, keepdims=True)
        ix = jnp.min(jnp.where(dw == mn, col + wnd * WINDOW, NUM_E), axis=-1)
        lvs.append(mn[:, 0])
        ivs.append(ix)

    pvs, pis = [], []
    for k in range(4):
        c = lvs[2 * k + 1] < lvs[2 * k]
        pvs.append(jnp.where(c, lvs[2 * k + 1], lvs[2 * k]))
        pis.append(jnp.where(c, ivs[2 * k + 1], ivs[2 * k]))

    acc_v = pvs[0].astype(jnp.bfloat16).astype(jnp.float32)
    acc_i = pis[0]
    for k in range(1, 4):
        upd = pvs[k] < acc_v
        acc_v = jnp.where(upd, pvs[k].astype(jnp.bfloat16).astype(jnp.float32),
                          acc_v)
        acc_i = jnp.where(upd, pis[k], acc_i)

    idx_ref[0, 0, :] = acc_i.astype(jnp.int32)


def _propose(flat, w, a, b):
    n_blocks = NUM_E // ROW_BLOCK
    idx3 = pl.pallas_call(
        _argmin_body,
        grid=(n_blocks,),
        in_specs=[
            pl.BlockSpec((ROW_BLOCK, DIM), lambda i: (i, 0)),
            pl.BlockSpec((NUM_E, DIM), lambda i: (0, 0)),
            pl.BlockSpec((ROW_BLOCK, 1), lambda i: (i, 0)),
            pl.BlockSpec((1, NUM_E), lambda i: (0, 0)),
        ],
        out_specs=pl.BlockSpec((1, 1, ROW_BLOCK), lambda i: (i, 0, 0)),
        out_shape=jax.ShapeDtypeStruct((n_blocks, 1, ROW_BLOCK), jnp.int32),
        compiler_params=pltpu.CompilerParams(
            dimension_semantics=("arbitrary",),
        ),
    )(flat, w, a, b)
    return idx3.reshape(-1)


@functools.cache
def _make_sc_gather():
    @functools.partial(
        pl.kernel,
        out_type=jax.ShapeDtypeStruct((NUM_E, DIM), jnp.float32),
        mesh=plsc.VectorSubcoreMesh(core_axis_name="c", subcore_axis_name="s"),
        scratch_types=[
            pltpu.VMEM((_CHUNK,), jnp.int32),
            pltpu.VMEM((_CHUNK, DIM), jnp.float32),
            pltpu.SemaphoreType.DMA,
        ],
        compiler_params=pltpu.CompilerParams(use_tc_tiling_on_sc=False),
    )
    def _sc_gather(w_hbm, idx_hbm, out_hbm, idx_v, rows_v, sem):
        wid = lax.axis_index("s") * _NC + lax.axis_index("c")
        for j in range(_CHUNKS_PER_W):
            row = wid * _CHUNKS_PER_W + j
            pltpu.sync_copy(idx_hbm.at[row], idx_v)
            pltpu.async_copy(w_hbm.at[idx_v], rows_v, sem).wait()
            pltpu.sync_copy(rows_v, out_hbm.at[pl.ds(row * _CHUNK, _CHUNK)])

    return _sc_gather


def kernel(input, W):
    shape = input.shape
    flat = input.reshape(-1, shape[-1])
    a = jnp.sum(input * input, axis=2).reshape(-1, 1)
    b = jnp.sum(W * W, axis=1).reshape(1, -1)
    proposal = _propose(flat, W, a, b)
    idx2 = proposal.reshape(NUM_E // _CHUNK, _CHUNK)
    quantized = _make_sc_gather()(W, idx2)
    return (quantized.reshape(shape), proposal.reshape(shape[:-1]))


# speedup vs baseline: 1.0080x; 1.0080x over previous
import functools

import jax
import jax.numpy as jnp
from jax import lax
from jax.experimental import pallas as pl
from jax.experimental.pallas import tpu as pltpu
from jax.experimental.pallas import tpu_sc as plsc

NUM_E = 8192
DIM = 32
ROW_BLOCK = 256
WINDOW = 1024

_NC = 2
_NS = 16
_NW = _NC * _NS
_CHUNK = 128
_CHUNKS_PER_W = NUM_E // (_NW * _CHUNK)


def _argmin_body(x_ref, w_ref, a_ref, b_ref, idx_ref):
    x = x_ref[...]
    xb = (2.0 * x).astype(jnp.bfloat16)
    a = a_ref[...]

    lvs, ivs = [], []
    col = lax.broadcasted_iota(jnp.int32, (ROW_BLOCK, WINDOW), 1)
    for wnd in range(NUM_E // WINDOW):
        wb = w_ref[pl.ds(wnd * WINDOW, WINDOW), :].astype(jnp.bfloat16)
        m = lax.dot_general(xb, wb, (((1,), (1,)), ((), ())),
                            preferred_element_type=jnp.float32)
        dw = (a + b_ref[:, pl.ds(wnd * WINDOW, WINDOW)]) - m
        mn = jnp.min(dw, axis=-1, keepdims=True)
        ix = jnp.min(jnp.where(dw == mn, col, NUM_E), axis=-1) + wnd * WINDOW
        lvs.append(mn[:, 0])
        ivs.append(ix)

    pvs, pis = [], []
    for k in range(4):
        c = lvs[2 * k + 1] < lvs[2 * k]
        pvs.append(jnp.where(c, lvs[2 * k + 1], lvs[2 * k]))
        pis.append(jnp.where(c, ivs[2 * k + 1], ivs[2 * k]))

    acc_v = pvs[0].astype(jnp.bfloat16).astype(jnp.float32)
    acc_i = pis[0]
    for k in range(1, 4):
        upd = pvs[k] < acc_v
        acc_v = jnp.where(upd, pvs[k].astype(jnp.bfloat16).astype(jnp.float32),
                          acc_v)
        acc_i = jnp.where(upd, pis[k], acc_i)

    idx_ref[0, 0, :] = acc_i.astype(jnp.int32)


def _propose(flat, w, a, b):
    n_blocks = NUM_E // ROW_BLOCK
    idx3 = pl.pallas_call(
        _argmin_body,
        grid=(n_blocks,),
        in_specs=[
            pl.BlockSpec((ROW_BLOCK, DIM), lambda i: (i, 0)),
            pl.BlockSpec((NUM_E, DIM), lambda i: (0, 0)),
            pl.BlockSpec((ROW_BLOCK, 1), lambda i: (i, 0)),
            pl.BlockSpec((1, NUM_E), lambda i: (0, 0)),
        ],
        out_specs=pl.BlockSpec((1, 1, ROW_BLOCK), lambda i: (i, 0, 0)),
        out_shape=jax.ShapeDtypeStruct((n_blocks, 1, ROW_BLOCK), jnp.int32),
        compiler_params=pltpu.CompilerParams(
            dimension_semantics=("arbitrary",),
        ),
    )(flat, w, a, b)
    return idx3.reshape(-1)


@functools.cache
def _make_sc_gather():
    @functools.partial(
        pl.kernel,
        out_type=jax.ShapeDtypeStruct((NUM_E, DIM), jnp.float32),
        mesh=plsc.VectorSubcoreMesh(core_axis_name="c", subcore_axis_name="s"),
        scratch_types=[
            pltpu.VMEM((_CHUNK,), jnp.int32),
            pltpu.VMEM((_CHUNK, DIM), jnp.float32),
            pltpu.SemaphoreType.DMA,
        ],
        compiler_params=pltpu.CompilerParams(use_tc_tiling_on_sc=False),
    )
    def _sc_gather(w_hbm, idx_hbm, out_hbm, idx_v, rows_v, sem):
        wid = lax.axis_index("s") * _NC + lax.axis_index("c")
        for j in range(_CHUNKS_PER_W):
            row = wid * _CHUNKS_PER_W + j
            pltpu.sync_copy(idx_hbm.at[row], idx_v)
            pltpu.async_copy(w_hbm.at[idx_v], rows_v, sem).wait()
            pltpu.sync_copy(rows_v, out_hbm.at[pl.ds(row * _CHUNK, _CHUNK)])

    return _sc_gather


def kernel(input, W):
    shape = input.shape
    flat = input.reshape(-1, shape[-1])
    a = jnp.sum(input * input, axis=2).reshape(-1, 1)
    b = jnp.sum(W * W, axis=1).reshape(1, -1)
    proposal = _propose(flat, W, a, b)
    idx2 = proposal.reshape(NUM_E // _CHUNK, _CHUNK)
    quantized = _make_sc_gather()(W, idx2)
    return (quantized.reshape(shape), proposal.reshape(shape[:-1]))


# speedup vs baseline: 1.0719x; 1.0633x over previous
import functools

import jax
import jax.numpy as jnp
from jax import lax
from jax.experimental import pallas as pl
from jax.experimental.pallas import tpu as pltpu
from jax.experimental.pallas import tpu_sc as plsc

NUM_E = 8192
DIM = 32
ROW_BLOCK = 512
WINDOW = 1024

_NC = 2
_NS = 16
_NW = _NC * _NS
_CHUNK = 128
_CHUNKS_PER_W = NUM_E // (_NW * _CHUNK)


def _argmin_body(x_ref, w_ref, a_ref, b_ref, idx_ref):
    x = x_ref[...]
    xb = (2.0 * x).astype(jnp.bfloat16)
    a = a_ref[...]

    lvs, ivs = [], []
    col = lax.broadcasted_iota(jnp.int32, (ROW_BLOCK, WINDOW), 1)
    for wnd in range(NUM_E // WINDOW):
        wb = w_ref[pl.ds(wnd * WINDOW, WINDOW), :].astype(jnp.bfloat16)
        m = lax.dot_general(xb, wb, (((1,), (1,)), ((), ())),
                            preferred_element_type=jnp.float32)
        dw = (a + b_ref[:, pl.ds(wnd * WINDOW, WINDOW)]) - m
        mn = jnp.min(dw, axis=-1, keepdims=True)
        ix = jnp.min(jnp.where(dw == mn, col, NUM_E), axis=-1) + wnd * WINDOW
        lvs.append(mn[:, 0])
        ivs.append(ix)

    pvs, pis = [], []
    for k in range(4):
        c = lvs[2 * k + 1] < lvs[2 * k]
        pvs.append(jnp.where(c, lvs[2 * k + 1], lvs[2 * k]))
        pis.append(jnp.where(c, ivs[2 * k + 1], ivs[2 * k]))

    acc_v = pvs[0].astype(jnp.bfloat16).astype(jnp.float32)
    acc_i = pis[0]
    for k in range(1, 4):
        upd = pvs[k] < acc_v
        acc_v = jnp.where(upd, pvs[k].astype(jnp.bfloat16).astype(jnp.float32),
                          acc_v)
        acc_i = jnp.where(upd, pis[k], acc_i)

    idx_ref[0, 0, :] = acc_i.astype(jnp.int32)


def _propose(flat, w, a, b):
    n_blocks = NUM_E // ROW_BLOCK
    idx3 = pl.pallas_call(
        _argmin_body,
        grid=(n_blocks,),
        in_specs=[
            pl.BlockSpec((ROW_BLOCK, DIM), lambda i: (i, 0)),
            pl.BlockSpec((NUM_E, DIM), lambda i: (0, 0)),
            pl.BlockSpec((ROW_BLOCK, 1), lambda i: (i, 0)),
            pl.BlockSpec((1, NUM_E), lambda i: (0, 0)),
        ],
        out_specs=pl.BlockSpec((1, 1, ROW_BLOCK), lambda i: (i, 0, 0)),
        out_shape=jax.ShapeDtypeStruct((n_blocks, 1, ROW_BLOCK), jnp.int32),
        compiler_params=pltpu.CompilerParams(
            dimension_semantics=("parallel",),
        ),
    )(flat, w, a, b)
    return idx3.reshape(-1)


@functools.cache
def _make_sc_gather():
    @functools.partial(
        pl.kernel,
        out_type=jax.ShapeDtypeStruct((NUM_E, DIM), jnp.float32),
        mesh=plsc.VectorSubcoreMesh(core_axis_name="c", subcore_axis_name="s"),
        scratch_types=[
            pltpu.VMEM((_CHUNK,), jnp.int32),
            pltpu.VMEM((_CHUNK, DIM), jnp.float32),
            pltpu.SemaphoreType.DMA,
        ],
        compiler_params=pltpu.CompilerParams(use_tc_tiling_on_sc=False),
    )
    def _sc_gather(w_hbm, idx_hbm, out_hbm, idx_v, rows_v, sem):
        wid = lax.axis_index("s") * _NC + lax.axis_index("c")
        for j in range(_CHUNKS_PER_W):
            row = wid * _CHUNKS_PER_W + j
            pltpu.sync_copy(idx_hbm.at[row], idx_v)
            pltpu.async_copy(w_hbm.at[idx_v], rows_v, sem).wait()
            pltpu.sync_copy(rows_v, out_hbm.at[pl.ds(row * _CHUNK, _CHUNK)])

    return _sc_gather


def kernel(input, W):
    shape = input.shape
    flat = input.reshape(-1, shape[-1])
    a = jnp.sum(input * input, axis=2).reshape(-1, 1)
    b = jnp.sum(W * W, axis=1).reshape(1, -1)
    proposal = _propose(flat, W, a, b)
    idx2 = proposal.reshape(NUM_E // _CHUNK, _CHUNK)
    quantized = _make_sc_gather()(W, idx2)
    return (quantized.reshape(shape), proposal.reshape(shape[:-1]))
